# baseline (speedup 1.0000x reference)
import jax
import jax.numpy as jnp
from jax import lax
from jax.experimental import pallas as pl
from jax.experimental.pallas import tpu as pltpu

POS_IOU_THRES = 0.7
NEG_IOU_THRES = 0.3
N_SAMPLE = 256
N_POS_TARGET = float(N_SAMPLE // 2)

N_PAD = 20480
NBLK = 10240
NB = N_PAD // NBLK
G = 64

_INTERPRET = False


def _cumsum_lanes(x):
    n = x.shape[-1]
    r = n // 128
    y = x.reshape(r, 128)
    lane = lax.broadcasted_iota(jnp.int32, (r, 128), 1)
    k = 1
    while k < 128:
        y = y + jnp.where(lane >= k, pltpu.roll(y, k, axis=1),
                          jnp.zeros((), x.dtype))
        k *= 2
    tot = y[:, 127:128]
    sub = lax.broadcasted_iota(jnp.int32, (r, 1), 0)
    t = tot
    k = 1
    while k < r:
        t = t + jnp.where(sub >= k, pltpu.roll(t, k, axis=0),
                          jnp.zeros((), x.dtype))
        k *= 2
    y = y + (t - tot)
    return y.reshape(1, n)


def _tg_kernel(a_ref, gt_ref, gtl_ref, boxes_o, loc_o, lab_o, cls_o,
               iou_s, gtb_s, lab_s, nrank_s, match_s, carry_s):
    p = pl.program_id(1)
    nb = pl.program_id(2)
    ds = pl.ds(nb * NBLK, NBLK)

    @pl.when(p == 0)
    def _pass0():
        a = a_ref[0]
        ay1, ax1, ay2, ax2 = a[0:1], a[1:2], a[2:3], a[3:4]
        g = gt_ref[0]
        gy1, gx1, gy2, gx2 = g[:, 0:1], g[:, 1:2], g[:, 2:3], g[:, 3:4]
        ih = jnp.clip(jnp.minimum(ay2, gy2) - jnp.maximum(ay1, gy1), 0.0)
        iw = jnp.clip(jnp.minimum(ax2, gx2) - jnp.maximum(ax1, gx1), 0.0)
        inter = ih * iw
        area_a = jnp.clip(ay2 - ay1, 0.0) * jnp.clip(ax2 - ax1, 0.0)
        area_g = jnp.clip(gy2 - gy1, 0.0) * jnp.clip(gx2 - gx1, 0.0)
        iou = inter / (area_a + area_g - inter + 1e-8)
        iou_s[:, ds] = iou
        prev = jnp.where(nb == 0, jnp.zeros((G, 1), jnp.float32), gtb_s[...])
        gtb_s[...] = jnp.maximum(prev, jnp.max(iou, axis=1, keepdims=True))

    @pl.when(p == 1)
    def _pass1():
        iou = iou_s[:, ds]
        max_iou = jnp.max(iou, axis=0, keepdims=True)
        iota = lax.broadcasted_iota(jnp.int32, (G, NBLK), 0)
        gidx = jnp.min(jnp.where(iou == max_iou, iota, G),
                       axis=0, keepdims=True)
        onehot = (iota == gidx).astype(jnp.bfloat16)
        gl = gtl_ref[0]
        g24 = jnp.dot(gl, onehot, preferred_element_type=jnp.float32)
        gath = (g24[0:8] + g24[8:16]) + g24[16:24]
        by1, bx1, by2, bx2 = gath[0:1], gath[1:2], gath[2:3], gath[3:4]
        boxes_o[0] = gath[0:4]
        match_s[:, ds] = gath[4:5]
        a = a_ref[0]
        ay1, ax1, ay2, ax2 = a[0:1], a[1:2], a[2:3], a[3:4]
        ah = jnp.maximum(ay2 - ay1, 1e-6)
        aw = jnp.maximum(ax2 - ax1, 1e-6)
        acy = ay1 + 0.5 * ah
        acx = ax1 + 0.5 * aw
        gh = jnp.maximum(by2 - by1, 1e-6)
        gw = jnp.maximum(bx2 - bx1, 1e-6)
        gcy = by1 + 0.5 * gh
        gcx = bx1 + 0.5 * gw
        loc_o[0] = jnp.concatenate(
            [(gcy - acy) / ah, (gcx - acx) / aw,
             jnp.log(gh / ah), jnp.log(gw / aw)], axis=0)
        gtb = gtb_s[...]
        best = jnp.max(jnp.where((iou == gtb) & (gtb > 0.0), 1.0, 0.0),
                       axis=0, keepdims=True)
        label = jnp.where(max_iou < NEG_IOU_THRES, 0.0, -1.0)
        label = jnp.where(best > 0.0, 1.0, label)
        label = jnp.where(max_iou >= POS_IOU_THRES, 1.0, label)
        pos = label == 1.0
        neg = label == 0.0
        pack = (pos.astype(jnp.int32)
                + (neg.astype(jnp.int32) << 15))
        pc = jnp.where(nb == 0, 0, carry_s[0])
        cum = pc + _cumsum_lanes(pack)
        carry_s[0] = pc + jnp.sum(pack)
        prank = cum & 0x7FFF
        label = jnp.where(pos & (prank > N_SAMPLE // 2), -1.0, label)
        nrank_s[:, ds] = (cum >> 15).astype(jnp.float32)
        lab_s[:, ds] = label

    @pl.when(p == 2)
    def _pass2():
        n_pos = carry_s[0] & 0x7FFF
        n_neg = (float(N_SAMPLE)
                 - jnp.minimum(n_pos, N_SAMPLE // 2).astype(jnp.float32))
        label = lab_s[:, ds]
        nrank = nrank_s[:, ds]
        label = jnp.where((label == 0.0) & (nrank > n_neg), -1.0, label)
        lab_o[0] = label
        mlab = match_s[:, ds]
        clsf = jnp.where(label == 1.0, mlab + 1.0,
                         jnp.where(label == 0.0, 0.0, -1.0))
        cls_o[0] = clsf.astype(jnp.int32)


def kernel(anchors, gt_boxes, obj_labels):
    B, N, _ = anchors.shape
    a_t = jnp.transpose(anchors.astype(jnp.float32), (0, 2, 1))
    a_t = jnp.pad(a_t, ((0, 0), (0, 0), (0, N_PAD - N)))
    gt = gt_boxes.astype(jnp.float32)
    glf = jnp.concatenate([
        jnp.transpose(gt, (0, 2, 1)),
        obj_labels.astype(jnp.float32)[:, None, :],
        jnp.zeros((B, 3, G), jnp.float32)], axis=1)
    hi = glf.astype(jnp.bfloat16)
    rem = glf - hi.astype(jnp.float32)
    mid = rem.astype(jnp.bfloat16)
    lo = (rem - mid.astype(jnp.float32)).astype(jnp.bfloat16)
    gtl = jnp.concatenate(
        [hi, mid, lo, jnp.zeros((B, 8, G), jnp.bfloat16)], axis=1)
    boxes_t, loc_t, lab2, cls2 = pl.pallas_call(
        _tg_kernel,
        grid=(B, 3, NB),
        in_specs=[
            pl.BlockSpec((1, 4, NBLK),
                         lambda b, p, nb: (b, 0, jnp.where(p == 2, 0, nb))),
            pl.BlockSpec((1, G, 4), lambda b, p, nb: (b, 0, 0)),
            pl.BlockSpec((1, 32, G), lambda b, p, nb: (b, 0, 0)),
        ],
        out_specs=[
            pl.BlockSpec((1, 4, NBLK),
                         lambda b, p, nb: (b, 0, jnp.where(p == 1, nb,
                                                jnp.where(p == 0, 0, NB - 1)))),
            pl.BlockSpec((1, 4, NBLK),
                         lambda b, p, nb: (b, 0, jnp.where(p == 1, nb,
                                                jnp.where(p == 0, 0, NB - 1)))),
            pl.BlockSpec((1, 1, NBLK),
                         lambda b, p, nb: (b, 0, jnp.where(p == 2, nb, 0))),
            pl.BlockSpec((1, 1, NBLK),
                         lambda b, p, nb: (b, 0, jnp.where(p == 2, nb, 0))),
        ],
        out_shape=[
            jax.ShapeDtypeStruct((B, 4, N_PAD), jnp.float32),
            jax.ShapeDtypeStruct((B, 4, N_PAD), jnp.float32),
            jax.ShapeDtypeStruct((B, 1, N_PAD), jnp.float32),
            jax.ShapeDtypeStruct((B, 1, N_PAD), jnp.int32),
        ],
        scratch_shapes=[
            pltpu.VMEM((G, N_PAD), jnp.float32),
            pltpu.VMEM((G, 1), jnp.float32),
            pltpu.VMEM((1, N_PAD), jnp.float32),
            pltpu.VMEM((1, N_PAD), jnp.float32),
            pltpu.VMEM((1, N_PAD), jnp.float32),
            pltpu.SMEM((2,), jnp.int32),
        ],
        compiler_params=pltpu.CompilerParams(
            dimension_semantics=("parallel", "arbitrary", "arbitrary")),
        interpret=_INTERPRET,
    )(a_t, gt, gtl)
    boxes = jnp.transpose(boxes_t, (0, 2, 1))[:, :N]
    loc = jnp.transpose(loc_t, (0, 2, 1))[:, :N]
    label = lab2[:, 0, :N]
    cls_label = cls2[:, 0, :N]
    return boxes, loc, label, cls_label


# speedup vs baseline: 1.0215x; 1.0215x over previous
import jax
import jax.numpy as jnp
from jax import lax
from jax.experimental import pallas as pl
from jax.experimental.pallas import tpu as pltpu

POS_IOU_THRES = 0.7
NEG_IOU_THRES = 0.3
N_SAMPLE = 256
N_POS_TARGET = float(N_SAMPLE // 2)

N_PAD = 20480
NBLK = 10240
NB = N_PAD // NBLK
G = 64

_INTERPRET = False


def _cumsum_lanes(x):
    n = x.shape[-1]
    r = n // 128
    y = x.reshape(r, 128)
    lane = lax.broadcasted_iota(jnp.int32, (r, 128), 1)
    k = 1
    while k < 128:
        y = y + jnp.where(lane >= k, pltpu.roll(y, k, axis=1),
                          jnp.zeros((), x.dtype))
        k *= 2
    tot = y[:, 127:128]
    sub = lax.broadcasted_iota(jnp.int32, (r, 1), 0)
    t = tot
    k = 1
    while k < r:
        t = t + jnp.where(sub >= k, pltpu.roll(t, k, axis=0),
                          jnp.zeros((), x.dtype))
        k *= 2
    y = y + (t - tot)
    return y.reshape(1, n)


def _tg_kernel(a_ref, gt_ref, gtl_ref, boxes_o, loc_o, lab_o, cls_o,
               iou_s, gtb_s, lab_s, nrank_s, match_s, carry_s):
    p = pl.program_id(1)
    nb = pl.program_id(2)
    ds = pl.ds(nb * NBLK, NBLK)

    @pl.when(p == 0)
    def _pass0():
        a = a_ref[0]
        ay1, ax1, ay2, ax2 = a[0:1], a[1:2], a[2:3], a[3:4]
        g = gt_ref[0]
        gy1, gx1, gy2, gx2 = g[:, 0:1], g[:, 1:2], g[:, 2:3], g[:, 3:4]
        ih = jnp.clip(jnp.minimum(ay2, gy2) - jnp.maximum(ay1, gy1), 0.0)
        iw = jnp.clip(jnp.minimum(ax2, gx2) - jnp.maximum(ax1, gx1), 0.0)
        inter = ih * iw
        area_a = jnp.clip(ay2 - ay1, 0.0) * jnp.clip(ax2 - ax1, 0.0)
        area_g = jnp.clip(gy2 - gy1, 0.0) * jnp.clip(gx2 - gx1, 0.0)
        iou = inter / (area_a + area_g - inter + 1e-8)
        iou_s[:, ds] = iou
        prev = jnp.where(nb == 0, jnp.zeros((G, 1), jnp.float32), gtb_s[...])
        gtb_s[...] = jnp.maximum(prev, jnp.max(iou, axis=1, keepdims=True))

    @pl.when(p == 1)
    def _pass1():
        iou = iou_s[:, ds]
        max_iou = jnp.max(iou, axis=0, keepdims=True)
        iota = lax.broadcasted_iota(jnp.int32, (G, NBLK), 0)
        gidx = jnp.min(jnp.where(iou == max_iou, iota, G),
                       axis=0, keepdims=True)
        onehot = (iota == gidx).astype(jnp.float32)
        gl = gtl_ref[0]
        gath = jnp.dot(gl, onehot, preferred_element_type=jnp.float32,
                       precision=lax.Precision.HIGHEST)
        by1, bx1, by2, bx2 = gath[0:1], gath[1:2], gath[2:3], gath[3:4]
        boxes_o[0] = gath[0:4]
        match_s[:, ds] = gath[4:5]
        a = a_ref[0]
        ay1, ax1, ay2, ax2 = a[0:1], a[1:2], a[2:3], a[3:4]
        ah = jnp.maximum(ay2 - ay1, 1e-6)
        aw = jnp.maximum(ax2 - ax1, 1e-6)
        acy = ay1 + 0.5 * ah
        acx = ax1 + 0.5 * aw
        gh = jnp.maximum(by2 - by1, 1e-6)
        gw = jnp.maximum(bx2 - bx1, 1e-6)
        gcy = by1 + 0.5 * gh
        gcx = bx1 + 0.5 * gw
        loc_o[0] = jnp.concatenate(
            [(gcy - acy) / ah, (gcx - acx) / aw,
             jnp.log(gh / ah), jnp.log(gw / aw)], axis=0)
        gtb = gtb_s[...]
        best = jnp.max(jnp.where((iou == gtb) & (gtb > 0.0), 1.0, 0.0),
                       axis=0, keepdims=True)
        label = jnp.where(max_iou < NEG_IOU_THRES, 0.0, -1.0)
        label = jnp.where(best > 0.0, 1.0, label)
        label = jnp.where(max_iou >= POS_IOU_THRES, 1.0, label)
        pos = label == 1.0
        neg = label == 0.0
        pack = (pos.astype(jnp.int32)
                + (neg.astype(jnp.int32) << 15))
        pc = jnp.where(nb == 0, 0, carry_s[0])
        cum = pc + _cumsum_lanes(pack)
        carry_s[0] = pc + jnp.sum(pack)
        prank = cum & 0x7FFF
        label = jnp.where(pos & (prank > N_SAMPLE // 2), -1.0, label)
        nrank_s[:, ds] = (cum >> 15).astype(jnp.float32)
        lab_s[:, ds] = label

    @pl.when(p == 2)
    def _pass2():
        n_pos = carry_s[0] & 0x7FFF
        n_neg = (float(N_SAMPLE)
                 - jnp.minimum(n_pos, N_SAMPLE // 2).astype(jnp.float32))
        label = lab_s[:, ds]
        nrank = nrank_s[:, ds]
        label = jnp.where((label == 0.0) & (nrank > n_neg), -1.0, label)
        lab_o[0] = label
        mlab = match_s[:, ds]
        clsf = jnp.where(label == 1.0, mlab + 1.0,
                         jnp.where(label == 0.0, 0.0, -1.0))
        cls_o[0] = clsf.astype(jnp.int32)


def kernel(anchors, gt_boxes, obj_labels):
    B, N, _ = anchors.shape
    a_t = jnp.transpose(anchors.astype(jnp.float32), (0, 2, 1))
    a_t = jnp.pad(a_t, ((0, 0), (0, 0), (0, N_PAD - N)))
    gt = gt_boxes.astype(jnp.float32)
    gtl = jnp.concatenate([
        jnp.transpose(gt, (0, 2, 1)),
        obj_labels.astype(jnp.float32)[:, None, :],
        jnp.zeros((B, 3, G), jnp.float32)], axis=1)
    boxes_t, loc_t, lab2, cls2 = pl.pallas_call(
        _tg_kernel,
        grid=(B, 3, NB),
        in_specs=[
            pl.BlockSpec((1, 4, NBLK),
                         lambda b, p, nb: (b, 0, jnp.where(p == 2, 0, nb))),
            pl.BlockSpec((1, G, 4), lambda b, p, nb: (b, 0, 0)),
            pl.BlockSpec((1, 8, G), lambda b, p, nb: (b, 0, 0)),
        ],
        out_specs=[
            pl.BlockSpec((1, 4, NBLK),
                         lambda b, p, nb: (b, 0, jnp.where(p == 1, nb,
                                                jnp.where(p == 0, 0, NB - 1)))),
            pl.BlockSpec((1, 4, NBLK),
                         lambda b, p, nb: (b, 0, jnp.where(p == 1, nb,
                                                jnp.where(p == 0, 0, NB - 1)))),
            pl.BlockSpec((1, 1, NBLK),
                         lambda b, p, nb: (b, 0, jnp.where(p == 2, nb, 0))),
            pl.BlockSpec((1, 1, NBLK),
                         lambda b, p, nb: (b, 0, jnp.where(p == 2, nb, 0))),
        ],
        out_shape=[
            jax.ShapeDtypeStruct((B, 4, N_PAD), jnp.float32),
            jax.ShapeDtypeStruct((B, 4, N_PAD), jnp.float32),
            jax.ShapeDtypeStruct((B, 1, N_PAD), jnp.float32),
            jax.ShapeDtypeStruct((B, 1, N_PAD), jnp.int32),
        ],
        scratch_shapes=[
            pltpu.VMEM((G, N_PAD), jnp.float32),
            pltpu.VMEM((G, 1), jnp.float32),
            pltpu.VMEM((1, N_PAD), jnp.float32),
            pltpu.VMEM((1, N_PAD), jnp.float32),
            pltpu.VMEM((1, N_PAD), jnp.float32),
            pltpu.SMEM((2,), jnp.int32),
        ],
        compiler_params=pltpu.CompilerParams(
            dimension_semantics=("parallel", "arbitrary", "arbitrary")),
        interpret=_INTERPRET,
    )(a_t, gt, gtl)
    boxes = jnp.transpose(boxes_t, (0, 2, 1))[:, :N]
    loc = jnp.transpose(loc_t, (0, 2, 1))[:, :N]
    label = lab2[:, 0, :N]
    cls_label = cls2[:, 0, :N]
    return boxes, loc, label, cls_label


# speedup vs baseline: 1.0673x; 1.0448x over previous
import jax
import jax.numpy as jnp
from jax import lax
from jax.experimental import pallas as pl
from jax.experimental.pallas import tpu as pltpu

POS_IOU_THRES = 0.7
NEG_IOU_THRES = 0.3
N_SAMPLE = 256
N_POS_TARGET = float(N_SAMPLE // 2)

N_PAD = 20480
NBLK = 20480
NB = N_PAD // NBLK
G = 64

_INTERPRET = False


def _cumsum_lanes(x):
    n = x.shape[-1]
    r = n // 128
    y = x.reshape(r, 128)
    lane = lax.broadcasted_iota(jnp.int32, (r, 128), 1)
    k = 1
    while k < 128:
        y = y + jnp.where(lane >= k, pltpu.roll(y, k, axis=1),
                          jnp.zeros((), x.dtype))
        k *= 2
    tot = y[:, 127:128]
    sub = lax.broadcasted_iota(jnp.int32, (r, 1), 0)
    t = tot
    k = 1
    while k < r:
        t = t + jnp.where(sub >= k, pltpu.roll(t, k, axis=0),
                          jnp.zeros((), x.dtype))
        k *= 2
    y = y + (t - tot)
    return y.reshape(1, n)


def _tg_kernel(a_ref, gt_ref, gtl_ref, boxes_o, loc_o, lab_o, cls_o,
               iou_s, gtb_s, lab_s, nrank_s, match_s, carry_s):
    p = pl.program_id(1)
    nb = pl.program_id(2)
    ds = pl.ds(nb * NBLK, NBLK)

    @pl.when(p == 0)
    def _pass0():
        a = a_ref[0]
        ay1, ax1, ay2, ax2 = a[0:1], a[1:2], a[2:3], a[3:4]
        g = gt_ref[0]
        gy1, gx1, gy2, gx2 = g[:, 0:1], g[:, 1:2], g[:, 2:3], g[:, 3:4]
        ih = jnp.clip(jnp.minimum(ay2, gy2) - jnp.maximum(ay1, gy1), 0.0)
        iw = jnp.clip(jnp.minimum(ax2, gx2) - jnp.maximum(ax1, gx1), 0.0)
        inter = ih * iw
        area_a = jnp.clip(ay2 - ay1, 0.0) * jnp.clip(ax2 - ax1, 0.0)
        area_g = jnp.clip(gy2 - gy1, 0.0) * jnp.clip(gx2 - gx1, 0.0)
        iou = inter / (area_a + area_g - inter + 1e-8)
        iou_s[:, ds] = iou
        prev = jnp.where(nb == 0, jnp.zeros((G, 1), jnp.float32), gtb_s[...])
        gtb_s[...] = jnp.maximum(prev, jnp.max(iou, axis=1, keepdims=True))

    @pl.when(p == 1)
    def _pass1():
        iou = iou_s[:, ds]
        max_iou = jnp.max(iou, axis=0, keepdims=True)
        iota = lax.broadcasted_iota(jnp.int32, (G, NBLK), 0)
        gidx = jnp.min(jnp.where(iou == max_iou, iota, G),
                       axis=0, keepdims=True)
        onehot = (iota == gidx).astype(jnp.float32)
        gl = gtl_ref[0]
        gath = jnp.dot(gl, onehot, preferred_element_type=jnp.float32,
                       precision=lax.Precision.HIGHEST)
        by1, bx1, by2, bx2 = gath[0:1], gath[1:2], gath[2:3], gath[3:4]
        boxes_o[0] = gath[0:4]
        match_s[:, ds] = gath[4:5]
        a = a_ref[0]
        ay1, ax1, ay2, ax2 = a[0:1], a[1:2], a[2:3], a[3:4]
        ah = jnp.maximum(ay2 - ay1, 1e-6)
        aw = jnp.maximum(ax2 - ax1, 1e-6)
        acy = ay1 + 0.5 * ah
        acx = ax1 + 0.5 * aw
        gh = jnp.maximum(by2 - by1, 1e-6)
        gw = jnp.maximum(bx2 - bx1, 1e-6)
        gcy = by1 + 0.5 * gh
        gcx = bx1 + 0.5 * gw
        loc_o[0] = jnp.concatenate(
            [(gcy - acy) / ah, (gcx - acx) / aw,
             jnp.log(gh / ah), jnp.log(gw / aw)], axis=0)
        gtb = gtb_s[...]
        best = jnp.max(jnp.where((iou == gtb) & (gtb > 0.0), 1.0, 0.0),
                       axis=0, keepdims=True)
        label = jnp.where(max_iou < NEG_IOU_THRES, 0.0, -1.0)
        label = jnp.where(best > 0.0, 1.0, label)
        label = jnp.where(max_iou >= POS_IOU_THRES, 1.0, label)
        pos = label == 1.0
        neg = label == 0.0
        pack = (pos.astype(jnp.int32)
                + (neg.astype(jnp.int32) << 15))
        pc = jnp.where(nb == 0, 0, carry_s[0])
        cum = pc + _cumsum_lanes(pack)
        carry_s[0] = pc + jnp.sum(pack)
        prank = cum & 0x7FFF
        label = jnp.where(pos & (prank > N_SAMPLE // 2), -1.0, label)
        nrank_s[:, ds] = (cum >> 15).astype(jnp.float32)
        lab_s[:, ds] = label

    @pl.when(p == 2)
    def _pass2():
        n_pos = carry_s[0] & 0x7FFF
        n_neg = (float(N_SAMPLE)
                 - jnp.minimum(n_pos, N_SAMPLE // 2).astype(jnp.float32))
        label = lab_s[:, ds]
        nrank = nrank_s[:, ds]
        label = jnp.where((label == 0.0) & (nrank > n_neg), -1.0, label)
        lab_o[0] = label
        mlab = match_s[:, ds]
        clsf = jnp.where(label == 1.0, mlab + 1.0,
                         jnp.where(label == 0.0, 0.0, -1.0))
        cls_o[0] = clsf.astype(jnp.int32)


def kernel(anchors, gt_boxes, obj_labels):
    B, N, _ = anchors.shape
    a_t = jnp.transpose(anchors.astype(jnp.float32), (0, 2, 1))
    a_t = jnp.pad(a_t, ((0, 0), (0, 0), (0, N_PAD - N)))
    gt = gt_boxes.astype(jnp.float32)
    gtl = jnp.concatenate([
        jnp.transpose(gt, (0, 2, 1)),
        obj_labels.astype(jnp.float32)[:, None, :],
        jnp.zeros((B, 3, G), jnp.float32)], axis=1)
    boxes_t, loc_t, lab2, cls2 = pl.pallas_call(
        _tg_kernel,
        grid=(B, 3, NB),
        in_specs=[
            pl.BlockSpec((1, 4, NBLK),
                         lambda b, p, nb: (b, 0, jnp.where(p == 2, 0, nb))),
            pl.BlockSpec((1, G, 4), lambda b, p, nb: (b, 0, 0)),
            pl.BlockSpec((1, 8, G), lambda b, p, nb: (b, 0, 0)),
        ],
        out_specs=[
            pl.BlockSpec((1, 4, NBLK),
                         lambda b, p, nb: (b, 0, jnp.where(p == 1, nb,
                                                jnp.where(p == 0, 0, NB - 1)))),
            pl.BlockSpec((1, 4, NBLK),
                         lambda b, p, nb: (b, 0, jnp.where(p == 1, nb,
                                                jnp.where(p == 0, 0, NB - 1)))),
            pl.BlockSpec((1, 1, NBLK),
                         lambda b, p, nb: (b, 0, jnp.where(p == 2, nb, 0))),
            pl.BlockSpec((1, 1, NBLK),
                         lambda b, p, nb: (b, 0, jnp.where(p == 2, nb, 0))),
        ],
        out_shape=[
            jax.ShapeDtypeStruct((B, 4, N_PAD), jnp.float32),
            jax.ShapeDtypeStruct((B, 4, N_PAD), jnp.float32),
            jax.ShapeDtypeStruct((B, 1, N_PAD), jnp.float32),
            jax.ShapeDtypeStruct((B, 1, N_PAD), jnp.int32),
        ],
        scratch_shapes=[
            pltpu.VMEM((G, N_PAD), jnp.float32),
            pltpu.VMEM((G, 1), jnp.float32),
            pltpu.VMEM((1, N_PAD), jnp.float32),
            pltpu.VMEM((1, N_PAD), jnp.float32),
            pltpu.VMEM((1, N_PAD), jnp.float32),
            pltpu.SMEM((2,), jnp.int32),
        ],
        compiler_params=pltpu.CompilerParams(
            dimension_semantics=("parallel", "arbitrary", "arbitrary")),
        interpret=_INTERPRET,
    )(a_t, gt, gtl)
    boxes = jnp.transpose(boxes_t, (0, 2, 1))[:, :N]
    loc = jnp.transpose(loc_t, (0, 2, 1))[:, :N]
    label = lab2[:, 0, :N]
    cls_label = cls2[:, 0, :N]
    return boxes, loc, label, cls_label
